# baseline (speedup 1.0000x reference)
import dataclasses
import functools

import jax
import jax.numpy as jnp
import numpy as np
from jax import lax
from jax.experimental import pallas as pl
from jax.experimental.pallas import tpu as pltpu
from jax.experimental.pallas import tpu_sc as plsc

LANES = 16
N_CORES = 2
N_SUBCORES = 16
NW = N_CORES * N_SUBCORES
NB = 128

_ATAN_C = (
    0.99999994,
    -0.33332303,
    0.19973682,
    -0.1404014,
    0.09967924,
    -0.06021913,
    0.02475678,
    -0.00483117,
)
_HALF_PI = np.float32(np.pi / 2)
_PI = np.float32(np.pi)

POSW = 16
B = 400


def _sc_partials(pos4, mi, mj, mk, mb, params):
    E = mi.shape[0]
    P = params.shape[1]
    T = round(P ** (1.0 / 3.0))
    chunk = E // NW
    nblocks = chunk // B
    assert nblocks * B == chunk and nblocks % 2 == 0
    tf32 = np.float32(T)

    mesh = plsc.VectorSubcoreMesh(core_axis_name="c", subcore_axis_name="s",
                                  num_cores=N_CORES, num_subcores=N_SUBCORES)
    cp = pltpu.CompilerParams()
    if "needs_layout_passes" in pltpu.CompilerParams.__dataclass_fields__:
        cp = dataclasses.replace(cp, needs_layout_passes=False)
    if "use_tc_tiling_on_sc" in pltpu.CompilerParams.__dataclass_fields__:
        cp = dataclasses.replace(cp, use_tc_tiling_on_sc=False)

    UNROLL = 5
    ACCW = UNROLL * LANES
    idx_t = pltpu.VMEM((B,), jnp.int32)
    row_t = pltpu.VMEM((B, POSW), jnp.float32)

    @functools.partial(
        pl.kernel,
        out_type=jax.ShapeDtypeStruct((NW, NB, LANES), jnp.float32),
        mesh=mesh,
        compiler_params=cp,
        scratch_types=[
            pltpu.VMEM((7, P), jnp.float32),
            pltpu.VMEM((NB, LANES), jnp.float32),
            [idx_t] * 8,
            [row_t] * 6,
            [pltpu.SemaphoreType.DMA] * 4,
        ],
    )
    def sck(pos4_hbm, mi_hbm, mj_hbm, mk_hbm, mb_hbm, params_hbm, out_hbm,
            params_v, acc_v, idxs, rows, sems):
        wid = lax.axis_index("s") * N_CORES + lax.axis_index("c")
        pltpu.sync_copy(params_hbm, params_v)

        zeros = jnp.zeros((LANES,), jnp.float32)

        @pl.loop(0, NB)
        def _(r):
            acc_v[r, :] = zeros

        base0 = wid * chunk
        ii = (idxs[0], idxs[4])
        ij = (idxs[1], idxs[5])
        ik = (idxs[2], idxs[6])
        bid = (idxs[3], idxs[7])
        ri = (rows[0], rows[3])
        rj = (rows[1], rows[4])
        rk = (rows[2], rows[5])
        sl = (sems[0], sems[1])
        sg = (sems[2], sems[3])

        def lin_issue(g, p):
            base = base0 + g * B
            pltpu.async_copy(mi_hbm.at[pl.ds(base, B)], ii[p], sl[p])
            pltpu.async_copy(mj_hbm.at[pl.ds(base, B)], ij[p], sl[p])
            pltpu.async_copy(mk_hbm.at[pl.ds(base, B)], ik[p], sl[p])

        def lin_wait(p):
            for dst in (ii[p], ij[p], ik[p]):
                pltpu.make_async_copy(mi_hbm.at[pl.ds(base0, B)], dst,
                                      sl[p]).wait()

        def gat_issue(g, p):
            base = base0 + g * B
            pltpu.async_copy(pos4_hbm.at[ii[p]], ri[p], sg[p])
            pltpu.async_copy(pos4_hbm.at[ij[p]], rj[p], sg[p])
            pltpu.async_copy(pos4_hbm.at[ik[p]], rk[p], sg[p])
            pltpu.async_copy(mb_hbm.at[pl.ds(base, B)], bid[p], sg[p])

        def gat_wait(p):
            pltpu.make_async_copy(pos4_hbm.at[ii[p]], ri[p], sg[p]).wait()
            pltpu.make_async_copy(pos4_hbm.at[ij[p]], rj[p], sg[p]).wait()
            pltpu.make_async_copy(pos4_hbm.at[ik[p]], rk[p], sg[p]).wait()
            pltpu.make_async_copy(mb_hbm.at[pl.ds(base0, B)], bid[p],
                                  sg[p]).wait()

        def compute(p):
            @plsc.parallel_loop(0, B, step=LANES, unroll=UNROLL)
            def _(v):
                lane = lax.iota(jnp.int32, LANES)
                lm = lane & 3
                colv = [(lane + m) & 3 for m in range(4)]
                msk = [lm == r for r in range(4)]
                row = v + lane

                def diag(ref):
                    return [plsc.load_gather(ref, [row, colv[m]])
                            for m in range(4)]

                pi = diag(ri[p])
                pj = diag(rj[p])
                pk = diag(rk[p])
                bv = bid[p][pl.ds(v, LANES)]

                def t_of(d):
                    return jnp.where(
                        msk[3], d[0],
                        jnp.where(msk[2], d[1],
                                  jnp.where(msk[1], d[2], d[3])))

                ti = t_of(pi)
                tj = t_of(pj)
                tk = t_of(pk)

                v1 = [pi[m] - pj[m] for m in range(4)]
                v2 = [pk[m] - pj[m] for m in range(4)]
                s11 = ((v1[0] * v1[0] + v1[1] * v1[1])
                       + (v1[2] * v1[2] + v1[3] * v1[3]))
                s12 = ((v1[0] * v2[0] + v1[1] * v2[1])
                       + (v1[2] * v2[2] + v1[3] * v2[3]))
                s22 = ((v2[0] * v2[0] + v2[1] * v2[1])
                       + (v2[2] * v2[2] + v2[3] * v2[3]))
                dt1 = ti - tj
                dt2 = tk - tj
                s11 = s11 - dt1 * dt1
                s12 = s12 - dt1 * dt2
                s22 = s22 - dt2 * dt2
                sin2 = jnp.maximum(s11 * s22 - s12 * s12, np.float32(0.0))
                cosd = s12

                ibits = plsc.bitcast(sin2, jnp.int32)
                magic = jnp.full((LANES,), 0x5F3759DF, jnp.int32)
                yb = plsc.bitcast(
                    magic - lax.shift_right_logical(ibits, 1), jnp.float32)
                half = sin2 * np.float32(0.5)
                yb = yb * (np.float32(1.5) - half * yb * yb)
                yb = yb * (np.float32(1.5) - half * yb * yb)
                yb = yb * (np.float32(1.5) - half * yb * yb)
                sin_t = jnp.where(sin2 > 0, sin2 * yb, np.float32(0.0))

                ac = jnp.abs(cosd)
                mx = jnp.maximum(sin_t, ac)
                mn = jnp.minimum(sin_t, ac)
                den = jnp.where(mx > 0, mx, np.float32(1.0))
                t = mn / den
                t2 = t * t
                poly = jnp.full((LANES,), _ATAN_C[-1], jnp.float32)
                for cc in _ATAN_C[-2::-1]:
                    poly = poly * t2 + np.float32(cc)
                a = poly * t
                a = jnp.where(sin_t > ac, _HALF_PI - a, a)
                x = jnp.where(cosd < np.float32(0.0), _PI - a, a)

                tidx = ((ti * tf32 + tj) * tf32 + tk).astype(jnp.int32)

                def par(r):
                    rowv = jnp.full((LANES,), r, jnp.int32)
                    return plsc.load_gather(params_v, [rowv, tidx])

                k0 = par(0)
                k1 = par(1)
                k2 = par(2)
                x00 = par(3)
                x01 = par(4)
                x02 = par(5)
                v0e = par(6)

                d0 = x - x00
                d1 = x - x01
                d2 = x - x02
                d2sq = d2 * d2
                Ve = (v0e + k0 * (d0 * d0) + k1 * (d1 * d1 * d1)
                      + k2 * (d2sq * d2sq))

                plsc.addupdate_scatter(acc_v, [bv, lane], Ve)

        def halfstep(g, p):
            q = 1 - p
            lin_wait(p)
            gat_issue(g, p)

            @pl.when(g >= 1)
            def _():
                gat_wait(q)
                compute(q)

            @pl.when(g + 1 < nblocks)
            def _():
                lin_issue(g + 1, q)

        lin_issue(0, 0)

        @pl.loop(0, nblocks // 2)
        def _(i):
            g = i * 2
            halfstep(g, 0)
            halfstep(g + 1, 1)

        gat_wait(1)
        compute(1)

        pltpu.sync_copy(acc_v, out_hbm.at[wid])

    return sck(pos4, mi, mj, mk, mb, params)


def _tc_reduce(partials):
    def body(x_ref, o_ref):
        o_ref[...] = jnp.sum(x_ref[...], axis=(0, 2))

    return pl.pallas_call(
        body,
        out_shape=jax.ShapeDtypeStruct((NB,), jnp.float32),
    )(partials)


@jax.jit
def kernel(pos, mapping, mapping_batch, atom_types, ks, x0s, v_0):
    n = pos.shape[0]
    pos4 = jnp.concatenate(
        [pos, atom_types.astype(jnp.float32)[:, None],
         jnp.zeros((n, POSW - 4), jnp.float32)], axis=1)
    params = jnp.concatenate(
        [ks.reshape(3, -1), x0s.reshape(3, -1), v_0.reshape(1, -1)], axis=0)
    mi = mapping[0]
    mj = mapping[1]
    mk = mapping[2]
    partials = _sc_partials(pos4, mi, mj, mk, mapping_batch, params)
    return _tc_reduce(partials)


# speedup vs baseline: 1.0963x; 1.0963x over previous
import dataclasses
import functools

import jax
import jax.numpy as jnp
import numpy as np
from jax import lax
from jax.experimental import pallas as pl
from jax.experimental.pallas import tpu as pltpu
from jax.experimental.pallas import tpu_sc as plsc

LANES = 16
N_CORES = 2
N_SUBCORES = 16
NW = N_CORES * N_SUBCORES
NB = 128

_ATAN_C = (
    0.99999994,
    -0.33332303,
    0.19973682,
    -0.1404014,
    0.09967924,
    -0.06021913,
    0.02475678,
    -0.00483117,
)
_HALF_PI = np.float32(np.pi / 2)
_PI = np.float32(np.pi)

POSW = 16
B = 400


def _sc_partials(pos4, mi, mj, mk, mb, params):
    E = mi.shape[0]
    P = params.shape[1]
    T = round(P ** (1.0 / 3.0))
    chunk = E // NW
    nblocks = chunk // B
    assert nblocks * B == chunk and nblocks % 2 == 0
    tf32 = np.float32(T)

    mesh = plsc.VectorSubcoreMesh(core_axis_name="c", subcore_axis_name="s",
                                  num_cores=N_CORES, num_subcores=N_SUBCORES)
    cp = pltpu.CompilerParams()
    if "needs_layout_passes" in pltpu.CompilerParams.__dataclass_fields__:
        cp = dataclasses.replace(cp, needs_layout_passes=False)
    if "use_tc_tiling_on_sc" in pltpu.CompilerParams.__dataclass_fields__:
        cp = dataclasses.replace(cp, use_tc_tiling_on_sc=False)

    UNROLL = 5
    ACCW = UNROLL * LANES
    idx_t = pltpu.VMEM((B,), jnp.int32)
    row_t = pltpu.VMEM((B, POSW), jnp.float32)

    @functools.partial(
        pl.kernel,
        out_type=jax.ShapeDtypeStruct((NW, NB, ACCW), jnp.float32),
        mesh=mesh,
        compiler_params=cp,
        scratch_types=[
            pltpu.VMEM((7, P), jnp.float32),
            pltpu.VMEM((NB, ACCW), jnp.float32),
            [idx_t] * 8,
            [row_t] * 6,
            [pltpu.SemaphoreType.DMA] * 4,
        ],
    )
    def sck(pos4_hbm, mi_hbm, mj_hbm, mk_hbm, mb_hbm, params_hbm, out_hbm,
            params_v, acc_v, idxs, rows, sems):
        wid = lax.axis_index("s") * N_CORES + lax.axis_index("c")
        pltpu.sync_copy(params_hbm, params_v)

        zeros = jnp.zeros((LANES,), jnp.float32)

        @pl.loop(0, NB)
        def _(r):
            for u in range(UNROLL):
                acc_v[r, pl.ds(u * LANES, LANES)] = zeros

        base0 = wid * chunk
        ii = (idxs[0], idxs[4])
        ij = (idxs[1], idxs[5])
        ik = (idxs[2], idxs[6])
        bid = (idxs[3], idxs[7])
        ri = (rows[0], rows[3])
        rj = (rows[1], rows[4])
        rk = (rows[2], rows[5])
        sl = (sems[0], sems[1])
        sg = (sems[2], sems[3])

        def lin_issue(g, p):
            base = base0 + g * B
            pltpu.async_copy(mi_hbm.at[pl.ds(base, B)], ii[p], sl[p])
            pltpu.async_copy(mj_hbm.at[pl.ds(base, B)], ij[p], sl[p])
            pltpu.async_copy(mk_hbm.at[pl.ds(base, B)], ik[p], sl[p])

        def lin_wait(p):
            for dst in (ii[p], ij[p], ik[p]):
                pltpu.make_async_copy(mi_hbm.at[pl.ds(base0, B)], dst,
                                      sl[p]).wait()

        def gat_issue(g, p):
            base = base0 + g * B
            pltpu.async_copy(pos4_hbm.at[ii[p]], ri[p], sg[p])
            pltpu.async_copy(pos4_hbm.at[ij[p]], rj[p], sg[p])
            pltpu.async_copy(pos4_hbm.at[ik[p]], rk[p], sg[p])
            pltpu.async_copy(mb_hbm.at[pl.ds(base, B)], bid[p], sg[p])

        def gat_wait(p):
            pltpu.make_async_copy(pos4_hbm.at[ii[p]], ri[p], sg[p]).wait()
            pltpu.make_async_copy(pos4_hbm.at[ij[p]], rj[p], sg[p]).wait()
            pltpu.make_async_copy(pos4_hbm.at[ik[p]], rk[p], sg[p]).wait()
            pltpu.make_async_copy(mb_hbm.at[pl.ds(base0, B)], bid[p],
                                  sg[p]).wait()

        def compute(p):
            @pl.loop(0, B, step=UNROLL * LANES)
            def _(v0):
                lane = lax.iota(jnp.int32, LANES)
                lm = lane & 3
                colv = [(lane + m) & 3 for m in range(4)]
                msk = [lm == r for r in range(4)]

                loads = []
                bvs = []
                for u in range(UNROLL):
                    v = v0 + u * LANES
                    row = v + lane

                    def diag(ref):
                        return [plsc.load_gather(ref, [row, colv[m]])
                                for m in range(4)]

                    loads.append((diag(ri[p]), diag(rj[p]), diag(rk[p])))
                    bvs.append(bid[p][pl.ds(v, LANES)])

                ves = []
                for u in range(UNROLL):
                    pi, pj, pk = loads[u]

                    def t_of(d):
                        return jnp.where(
                            msk[3], d[0],
                            jnp.where(msk[2], d[1],
                                      jnp.where(msk[1], d[2], d[3])))

                    ti = t_of(pi)
                    tj = t_of(pj)
                    tk = t_of(pk)

                    v1 = [pi[m] - pj[m] for m in range(4)]
                    v2 = [pk[m] - pj[m] for m in range(4)]
                    s11 = ((v1[0] * v1[0] + v1[1] * v1[1])
                           + (v1[2] * v1[2] + v1[3] * v1[3]))
                    s12 = ((v1[0] * v2[0] + v1[1] * v2[1])
                           + (v1[2] * v2[2] + v1[3] * v2[3]))
                    s22 = ((v2[0] * v2[0] + v2[1] * v2[1])
                           + (v2[2] * v2[2] + v2[3] * v2[3]))
                    dt1 = ti - tj
                    dt2 = tk - tj
                    s11 = s11 - dt1 * dt1
                    s12 = s12 - dt1 * dt2
                    s22 = s22 - dt2 * dt2
                    sin2 = jnp.maximum(s11 * s22 - s12 * s12, np.float32(0.0))
                    cosd = s12

                    ibits = plsc.bitcast(sin2, jnp.int32)
                    magic = jnp.full((LANES,), 0x5F3759DF, jnp.int32)
                    yb = plsc.bitcast(
                        magic - lax.shift_right_logical(ibits, 1), jnp.float32)
                    half = sin2 * np.float32(0.5)
                    yb = yb * (np.float32(1.5) - half * yb * yb)
                    yb = yb * (np.float32(1.5) - half * yb * yb)
                    yb = yb * (np.float32(1.5) - half * yb * yb)
                    sin_t = jnp.where(sin2 > 0, sin2 * yb, np.float32(0.0))

                    ac = jnp.abs(cosd)
                    mx = jnp.maximum(sin_t, ac)
                    mn = jnp.minimum(sin_t, ac)
                    den = jnp.where(mx > 0, mx, np.float32(1.0))
                    t = mn / den
                    t2 = t * t
                    poly = jnp.full((LANES,), _ATAN_C[-1], jnp.float32)
                    for cc in _ATAN_C[-2::-1]:
                        poly = poly * t2 + np.float32(cc)
                    a = poly * t
                    a = jnp.where(sin_t > ac, _HALF_PI - a, a)
                    x = jnp.where(cosd < np.float32(0.0), _PI - a, a)

                    tidx = ((ti * tf32 + tj) * tf32 + tk).astype(jnp.int32)

                    def par(r):
                        rowv = jnp.full((LANES,), r, jnp.int32)
                        return plsc.load_gather(params_v, [rowv, tidx])

                    k0 = par(0)
                    k1 = par(1)
                    k2 = par(2)
                    x00 = par(3)
                    x01 = par(4)
                    x02 = par(5)
                    v0e = par(6)

                    d0 = x - x00
                    d1 = x - x01
                    d2 = x - x02
                    d2sq = d2 * d2
                    ves.append(v0e + k0 * (d0 * d0) + k1 * (d1 * d1 * d1)
                               + k2 * (d2sq * d2sq))

                for u in range(UNROLL):
                    plsc.addupdate_scatter(
                        acc_v, [bvs[u], lane + np.int32(u * LANES)], ves[u])

        def halfstep(g, p):
            q = 1 - p
            lin_wait(p)
            gat_issue(g, p)

            @pl.when(g >= 1)
            def _():
                gat_wait(q)
                compute(q)

            @pl.when(g + 1 < nblocks)
            def _():
                lin_issue(g + 1, q)

        lin_issue(0, 0)

        @pl.loop(0, nblocks // 2)
        def _(i):
            g = i * 2
            halfstep(g, 0)
            halfstep(g + 1, 1)

        gat_wait(1)
        compute(1)

        pltpu.sync_copy(acc_v, out_hbm.at[wid])

    return sck(pos4, mi, mj, mk, mb, params)


def _tc_reduce(partials):
    def body(x_ref, o_ref):
        o_ref[...] = jnp.sum(x_ref[...], axis=(0, 2))

    return pl.pallas_call(
        body,
        out_shape=jax.ShapeDtypeStruct((NB,), jnp.float32),
    )(partials)


@jax.jit
def kernel(pos, mapping, mapping_batch, atom_types, ks, x0s, v_0):
    n = pos.shape[0]
    pos4 = jnp.concatenate(
        [pos, atom_types.astype(jnp.float32)[:, None],
         jnp.zeros((n, POSW - 4), jnp.float32)], axis=1)
    params = jnp.concatenate(
        [ks.reshape(3, -1), x0s.reshape(3, -1), v_0.reshape(1, -1)], axis=0)
    mi = mapping[0]
    mj = mapping[1]
    mk = mapping[2]
    partials = _sc_partials(pos4, mi, mj, mk, mapping_batch, params)
    return _tc_reduce(partials)
